# Initial kernel scaffold; baseline (speedup 1.0000x reference)
import jax
import jax.numpy as jnp
from jax.experimental import pallas as pl


def kernel(x, edge_index, W2, u2, c2, b2, W3, u3, c3, b3, W4, u4, c4, b4, W5, u5, c5, b5, W6, u6, c6, b6, lin1_w, lin1_b, lin2_w, lin2_b, out_w, out_b):
    raise NotImplementedError("write your pallas kernel here")


# speedup vs baseline: 5.9466x; 5.9466x over previous
import jax
import jax.numpy as jnp
from jax import lax
from jax.experimental import pallas as pl
from jax.experimental.pallas import tpu as pltpu
from jax.experimental.pallas import tpu_sc as plsc

NC, NS, L = 2, 16, 16
NW = NC * NS
CHUNK = 128
NPAD = 102400
RPT = NPAD // NS


def _mesh():
    return plsc.VectorSubcoreMesh(core_axis_name="c", subcore_axis_name="s")


def _sc_count(dst, ones_hbm, z8_hbm):
    E = dst.shape[0]
    nchunks = E // CHUNK
    niter = (nchunks + NW - 1) // NW

    def body(dst_hbm, ones_ref, z_ref, out_hbm, dstbuf, ones, cacc):
        cid = lax.axis_index("c")
        sid = lax.axis_index("s")
        wid = sid * NC + cid
        base0 = sid * RPT
        pltpu.sync_copy(z_ref.at[pl.ds(0, RPT)], cacc.at[pl.ds(base0, RPT)])
        pltpu.sync_copy(ones_ref, ones)
        plsc.subcore_barrier()

        def step(i, c):
            g = i * NW + wid

            @pl.when(g < nchunks)
            def _():
                base = g * CHUNK
                pltpu.sync_copy(dst_hbm.at[pl.ds(base, CHUNK)], dstbuf)
                pltpu.sync_copy(ones, cacc.at[dstbuf], add=True)
            return c

        lax.fori_loop(0, niter, step, 0)
        plsc.subcore_barrier()
        pltpu.sync_copy(cacc.at[pl.ds(base0, RPT)],
                        out_hbm.at[cid, pl.ds(base0, RPT)])

    k = pl.kernel(
        body,
        out_type=jax.ShapeDtypeStruct((NC, NPAD, 8), jnp.float32),
        mesh=_mesh(),
        compiler_params=pltpu.CompilerParams(use_tc_tiling_on_sc=False),
        scratch_types=[
            pltpu.VMEM((CHUNK,), jnp.int32),
            pltpu.VMEM((CHUNK, 8), jnp.float32),
            pltpu.VMEM_SHARED((NPAD, 8), jnp.float32),
        ],
    )
    return k(dst, ones_hbm, z8_hbm)


def _sc_pass_h1(h_tab, src, dst, z_hbm):
    E = src.shape[0]
    nchunks = E // CHUNK
    niter = (nchunks + NW - 1) // NW

    def body(h_hbm, src_hbm, dst_hbm, z_ref, out_hbm,
             srcbuf, dstbuf, rows, acc, sem):
        cid = lax.axis_index("c")
        sid = lax.axis_index("s")
        wid = sid * NC + cid
        base0 = sid * RPT
        pltpu.sync_copy(z_ref.at[pl.ds(0, RPT)], acc.at[pl.ds(base0, RPT)])
        plsc.subcore_barrier()

        def step(i, c):
            g = i * NW + wid

            @pl.when(g < nchunks)
            def _():
                base = g * CHUNK
                pltpu.sync_copy(src_hbm.at[pl.ds(base, CHUNK)], srcbuf)
                pltpu.sync_copy(dst_hbm.at[pl.ds(base, CHUNK)], dstbuf)
                pltpu.async_copy(h_hbm.at[srcbuf], rows, sem).wait()
                pltpu.sync_copy(rows, acc.at[dstbuf], add=True)
            return c

        lax.fori_loop(0, niter, step, 0)
        plsc.subcore_barrier()
        pltpu.sync_copy(acc.at[pl.ds(base0, RPT)],
                        out_hbm.at[cid, pl.ds(base0, RPT)])

    k = pl.kernel(
        body,
        out_type=jax.ShapeDtypeStruct((NC, NPAD, 16), jnp.float32),
        mesh=_mesh(),
        compiler_params=pltpu.CompilerParams(use_tc_tiling_on_sc=False),
        scratch_types=[
            pltpu.VMEM((CHUNK,), jnp.int32),
            pltpu.VMEM((CHUNK,), jnp.int32),
            pltpu.VMEM((CHUNK, 16), jnp.float32),
            pltpu.VMEM_SHARED((NPAD, 16), jnp.float32),
            pltpu.SemaphoreType.DMA,
        ],
    )
    return k(h_tab, src, dst, z_hbm)


def _sc_pass_h4(p_tab, y_tab, src, dst, ctile, z_hbm):
    E = src.shape[0]
    nchunks = E // CHUNK
    niter = (nchunks + NW - 1) // NW

    def body(p_hbm, y_hbm, src_hbm, dst_hbm, c_hbm, z_ref, out_hbm,
             srcbuf, dstbuf, pa, pb, yb, msg, cbuf, acc, sem):
        cid = lax.axis_index("c")
        sid = lax.axis_index("s")
        wid = sid * NC + cid
        base0 = sid * RPT
        pltpu.sync_copy(z_ref.at[pl.ds(0, RPT)], acc.at[pl.ds(base0, RPT)])
        pltpu.sync_copy(c_hbm, cbuf)
        plsc.subcore_barrier()

        cvec = cbuf[...]

        def step(i, c):
            g = i * NW + wid

            @pl.when(g < nchunks)
            def _():
                base = g * CHUNK
                pltpu.sync_copy(src_hbm.at[pl.ds(base, CHUNK)], srcbuf)
                pltpu.sync_copy(dst_hbm.at[pl.ds(base, CHUNK)], dstbuf)
                c1 = pltpu.async_copy(p_hbm.at[srcbuf], pa, sem)
                c2 = pltpu.async_copy(p_hbm.at[dstbuf], pb, sem)
                c3 = pltpu.async_copy(y_hbm.at[srcbuf], yb, sem)
                c1.wait(); c2.wait(); c3.wait()

                def edge(e, cc):
                    ex = jnp.exp(pa[e, :] - pb[e, :] + cvec)
                    s = ex[0] + ex[1] + ex[2] + ex[3]
                    m = None
                    for h in range(4):
                        yv = yb[e, pl.ds(16 * h, 16)]
                        term = ex[h] * yv
                        m = term if m is None else m + term
                    msg[e, :] = m / s
                    return cc

                lax.fori_loop(0, CHUNK, edge, 0, unroll=8)
                pltpu.sync_copy(msg, acc.at[dstbuf], add=True)
            return c

        lax.fori_loop(0, niter, step, 0)
        plsc.subcore_barrier()
        pltpu.sync_copy(acc.at[pl.ds(base0, RPT)],
                        out_hbm.at[cid, pl.ds(base0, RPT)])

    k = pl.kernel(
        body,
        out_type=jax.ShapeDtypeStruct((NC, NPAD, 16), jnp.float32),
        mesh=_mesh(),
        compiler_params=pltpu.CompilerParams(use_tc_tiling_on_sc=False),
        scratch_types=[
            pltpu.VMEM((CHUNK,), jnp.int32),
            pltpu.VMEM((CHUNK,), jnp.int32),
            pltpu.VMEM((CHUNK, 16), jnp.float32),
            pltpu.VMEM((CHUNK, 16), jnp.float32),
            pltpu.VMEM((CHUNK, 64), jnp.float32),
            pltpu.VMEM((CHUNK, 16), jnp.float32),
            pltpu.VMEM((16,), jnp.float32),
            pltpu.VMEM_SHARED((NPAD, 16), jnp.float32),
            pltpu.SemaphoreType.DMA,
        ],
    )
    return k(p_tab, y_tab, src, dst, ctile, z_hbm)


RB = 2000


def _rows(d):
    return pl.BlockSpec((RB, d), lambda i: (i, 0))


def _full(shape):
    return pl.BlockSpec(shape, lambda i: tuple(0 for _ in shape))


def _tc_call(fn, n, ins, in_specs, out_shapes, out_specs):
    return pl.pallas_call(
        fn,
        grid=(n // RB,),
        in_specs=in_specs,
        out_specs=out_specs,
        out_shape=out_shapes,
    )(*ins)


def _wc(W, c, heads, out_ch):
    q = jax.nn.softmax(c.reshape(heads))
    return (W.reshape(W.shape[0], heads, out_ch) * q[None, :, None]).sum(axis=1)


def kernel(x, edge_index, W2, u2, c2, b2, W3, u3, c3, b3, W4, u4, c4, b4,
           W5, u5, c5, b5, W6, u6, c6, b6, lin1_w, lin1_b, lin2_w, lin2_b,
           out_w, out_b):
    n = x.shape[0]
    src = edge_index[0]
    dst = edge_index[1]
    z_hbm = jnp.zeros((RPT, 16), jnp.float32)
    z8_hbm = jnp.zeros((RPT, 8), jnp.float32)
    ones_hbm = jnp.ones((CHUNK, 8), jnp.float32)

    cnt_parts = _sc_count(dst, ones_hbm, z8_hbm)
    cnt0 = cnt_parts[0, :n, 0].reshape(n, 1)
    cnt1 = cnt_parts[1, :n, 0].reshape(n, 1)

    def tc1(x_ref, W_ref, u_ref, Wc_ref, y_ref, p_ref, s_ref):
        h0 = jnp.maximum(x_ref[...], 0.0)
        y_ref[...] = jnp.dot(h0, W_ref[...], preferred_element_type=jnp.float32)
        p_ref[...] = jnp.dot(h0, u_ref[...], preferred_element_type=jnp.float32)
        s_ref[...] = jnp.dot(h0, Wc_ref[...], preferred_element_type=jnp.float32)

    W2c = _wc(W2, c2, 4, 16)
    u2p = jnp.pad(u2, ((0, 0), (0, 12)))
    y2, p2, self2 = _tc_call(
        tc1, n, [x, W2, u2p, W2c],
        [_rows(16), _full((16, 64)), _full((16, 16)), _full((16, 16))],
        [jax.ShapeDtypeStruct((n, 64), jnp.float32),
         jax.ShapeDtypeStruct((n, 16), jnp.float32),
         jax.ShapeDtypeStruct((n, 16), jnp.float32)],
        [_rows(64), _rows(16), _rows(16)],
    )

    acc2 = _sc_pass_h4(p2, y2, src, dst, jnp.pad(c2, (0, 12)), z_hbm)

    def tc2(aa, ab, sf, ca, cb, b_ref, W_ref, u_ref, Wc_ref,
            y_ref, p_ref, s_ref, inv_ref):
        cnt = ca[...] + cb[...] + 1.0
        inv = 1.0 / cnt
        inv_ref[...] = inv
        h1 = jnp.maximum((aa[...] + ab[...] + sf[...]) * inv + b_ref[...], 0.0)
        y_ref[...] = jnp.dot(h1, W_ref[...], preferred_element_type=jnp.float32)
        p_ref[...] = jnp.dot(h1, u_ref[...], preferred_element_type=jnp.float32)
        s_ref[...] = jnp.dot(h1, Wc_ref[...], preferred_element_type=jnp.float32)

    W3c = _wc(W3, c3, 4, 16)
    u3p = jnp.pad(u3, ((0, 0), (0, 12)))
    y3, p3, self3, inv = _tc_call(
        tc2, n, [acc2[0, :n], acc2[1, :n], self2, cnt0, cnt1,
                 b2.reshape(1, 16), W3, u3p, W3c],
        [_rows(16), _rows(16), _rows(16), _rows(1), _rows(1),
         _full((1, 16)), _full((16, 64)), _full((16, 16)), _full((16, 16))],
        [jax.ShapeDtypeStruct((n, 64), jnp.float32),
         jax.ShapeDtypeStruct((n, 16), jnp.float32),
         jax.ShapeDtypeStruct((n, 16), jnp.float32),
         jax.ShapeDtypeStruct((n, 1), jnp.float32)],
        [_rows(64), _rows(16), _rows(16), _rows(1)],
    )

    acc3 = _sc_pass_h4(p3, y3, src, dst, jnp.pad(c3, (0, 12)), z_hbm)

    def tc3(aa, ab, sf, inv_ref, b_ref, h_ref):
        h_ref[...] = jnp.maximum(
            (aa[...] + ab[...] + sf[...]) * inv_ref[...] + b_ref[...], 0.0)

    h2 = _tc_call(
        tc3, n, [acc3[0, :n], acc3[1, :n], self3, inv, b3.reshape(1, 16)],
        [_rows(16), _rows(16), _rows(16), _rows(1), _full((1, 16))],
        jax.ShapeDtypeStruct((n, 16), jnp.float32),
        _rows(16),
    )

    s4 = _sc_pass_h1(h2, src, dst, z_hbm)

    def tc4(aa, ab, hp, inv_ref, W_ref, b_ref, h_ref):
        agg = (aa[...] + ab[...] + hp[...]) * inv_ref[...]
        h_ref[...] = jnp.maximum(
            jnp.dot(agg, W_ref[...], preferred_element_type=jnp.float32)
            + b_ref[...], 0.0)

    h3 = _tc_call(
        tc4, n, [s4[0, :n], s4[1, :n], h2, inv, W4, b4.reshape(1, 16)],
        [_rows(16), _rows(16), _rows(16), _rows(1), _full((16, 16)),
         _full((1, 16))],
        jax.ShapeDtypeStruct((n, 16), jnp.float32),
        _rows(16),
    )

    s5 = _sc_pass_h1(h3, src, dst, z_hbm)

    def tc5(aa, ab, hp, inv_ref, W_ref, b_ref, h_ref):
        agg = (aa[...] + ab[...] + hp[...]) * inv_ref[...]
        h_ref[...] = jnp.maximum(
            jnp.dot(agg, W_ref[...], preferred_element_type=jnp.float32)
            + b_ref[...], 0.0)

    h4 = _tc_call(
        tc5, n, [s5[0, :n], s5[1, :n], h3, inv, W5, b5.reshape(1, 32)],
        [_rows(16), _rows(16), _rows(16), _rows(1), _full((16, 32)),
         _full((1, 32))],
        jax.ShapeDtypeStruct((n, 32), jnp.float32),
        _rows(32),
    )

    h4a = h4[:, :16]
    h4b = h4[:, 16:]
    s6a = _sc_pass_h1(h4a, src, dst, z_hbm)
    s6b = _sc_pass_h1(h4b, src, dst, z_hbm)

    def tc6(sa0, sb0, sa1, sb1, ha, hb, inv_ref, W_ref, b_ref,
            l1w, l1b, l2w, l2b, ow, ob, o_ref):
        agg0 = (sa0[...] + sb0[...] + ha[...]) * inv_ref[...]
        agg1 = (sa1[...] + sb1[...] + hb[...]) * inv_ref[...]
        agg = jnp.concatenate([agg0, agg1], axis=1)
        h5 = jnp.maximum(
            jnp.dot(agg, W_ref[...], preferred_element_type=jnp.float32)
            + b_ref[...], 0.0)
        h6 = jnp.maximum(
            jnp.dot(h5, l1w[...], preferred_element_type=jnp.float32)
            + l1b[...], 0.0)
        h7 = jnp.maximum(
            jnp.dot(h6, l2w[...], preferred_element_type=jnp.float32)
            + l2b[...], 0.0)
        o = jnp.dot(h7, ow[...], preferred_element_type=jnp.float32) + ob[...]
        o_ref[...] = jax.nn.sigmoid(o)

    out = _tc_call(
        tc6, n,
        [s6a[0, :n], s6a[1, :n], s6b[0, :n], s6b[1, :n], h4a, h4b, inv,
         W6, b6.reshape(1, 64), lin1_w, lin1_b.reshape(1, 16),
         lin2_w, lin2_b.reshape(1, 4), out_w, out_b.reshape(1, 1)],
        [_rows(16), _rows(16), _rows(16), _rows(16), _rows(16), _rows(16),
         _rows(1), _full((32, 64)), _full((1, 64)), _full((64, 16)),
         _full((1, 16)), _full((16, 4)), _full((1, 4)), _full((4, 1)),
         _full((1, 1))],
        jax.ShapeDtypeStruct((n, 1), jnp.float32),
        _rows(1),
    )
    return out
